# baseline (speedup 1.0000x reference)
import functools

import jax
import jax.numpy as jnp
from jax import lax
from jax.experimental import pallas as pl
from jax.experimental.pallas import tpu as pltpu
from jax.experimental.pallas import tpu_sc as plsc

_NUM_CORES = 2
_NUM_SUBCORES = 16
_NUM_WORKERS = _NUM_CORES * _NUM_SUBCORES


_CHUNK = 16
_NBUF = 3


def kernel(x, position_embeddings):
    seq_len = x.shape[1]
    emb_dim = position_embeddings.shape[1]
    rows_per_w = seq_len // _NUM_WORKERS
    n_chunks = rows_per_w // _CHUNK

    @functools.partial(
        pl.kernel,
        out_type=jax.ShapeDtypeStruct((seq_len, emb_dim), position_embeddings.dtype),
        mesh=plsc.VectorSubcoreMesh(core_axis_name="c", subcore_axis_name="s"),
        scratch_types=(
            [pltpu.VMEM((_CHUNK, emb_dim), jnp.float32)] * _NBUF
            + [pltpu.SemaphoreType.DMA] * (2 * _NBUF)
        ),
    )
    def copy_rows(table_hbm, out_hbm, *scratch):
        bufs = scratch[:_NBUF]
        rsems = scratch[_NBUF:2 * _NBUF]
        wsems = scratch[2 * _NBUF:]
        wid = lax.axis_index("s") * _NUM_CORES + lax.axis_index("c")
        base = wid * rows_per_w

        def rd(i):
            b = i % _NBUF
            return pltpu.make_async_copy(
                table_hbm.at[pl.ds(base + i * _CHUNK, _CHUNK)], bufs[b], rsems[b])

        def wr(i):
            b = i % _NBUF
            return pltpu.make_async_copy(
                bufs[b], out_hbm.at[pl.ds(base + i * _CHUNK, _CHUNK)], wsems[b])

        for i in range(min(_NBUF, n_chunks)):
            rd(i).start()
        for i in range(n_chunks):
            if i + 1 < n_chunks and i + 1 >= _NBUF:
                wr(i + 1 - _NBUF).wait()
                rd(i + 1).start()
            rd(i).wait()
            wr(i).start()
        for i in range(max(0, n_chunks - _NBUF), n_chunks):
            wr(i).wait()

    return copy_rows(position_embeddings)[None]
